# baseline (speedup 1.0000x reference)
import functools

import jax
import jax.numpy as jnp
from jax import lax
from jax.experimental import pallas as pl
from jax.experimental.pallas import tpu as pltpu
from jax.experimental.pallas import tpu_sc as plsc

_NC = 2
_NS = 16
_L = 16
_NW = _NC * _NS
_GR = 8
_GPC = 4
_CROWS = _GR * _GPC
_NBUF = 2
_NACC = 8


@functools.lru_cache(maxsize=None)
def _build(nrows: int, ncols: int):
    assert nrows % (_NW * _CROWS * _NBUF) == 0 and ncols % (8 * _L) == 0
    rpw = nrows // _NW
    nchunks = rpw // _CROWS
    vpr = ncols // _L
    mesh = plsc.VectorSubcoreMesh(core_axis_name="c", subcore_axis_name="s")

    @functools.partial(
        pl.kernel,
        out_type=jax.ShapeDtypeStruct((_NW, _L), jnp.float32),
        mesh=mesh,
        scratch_types=[
            pltpu.VMEM((_NBUF, _GPC, _GR, ncols), jnp.float32),
            pltpu.VMEM((_NBUF, _GPC, _GR, ncols), jnp.float32),
            pltpu.VMEM((_L,), jnp.float32),
            pltpu.SemaphoreType.DMA,
            pltpu.SemaphoreType.DMA,
            pltpu.SemaphoreType.DMA,
            pltpu.SemaphoreType.DMA,
        ],
    )
    def masked_mse_partials(o_hbm, t_hbm, out_hbm, obuf, tbuf, accv,
                            so0, so1, st0, st1):
        osems = (so0, so1)
        tsems = (st0, st1)
        wid = lax.axis_index("s") * _NC + lax.axis_index("c")
        base = wid * rpw

        o3 = o_hbm.reshape(nrows // _GR, _GR, ncols)
        t3 = t_hbm.reshape(nrows // _GR, _GR, ncols)

        def start(ci, b):
            grp = pl.multiple_of((base + ci * _CROWS) // _GR, _GPC)
            src = pl.ds(grp, _GPC)
            pltpu.async_copy(o3.at[src], obuf.at[b], osems[b])
            pltpu.async_copy(t3.at[src], tbuf.at[b], tsems[b])

        def wait(b):
            pltpu.make_async_copy(
                o3.at[pl.ds(0, _GPC)], obuf.at[b], osems[b]).wait()
            pltpu.make_async_copy(
                t3.at[pl.ds(0, _GPC)], tbuf.at[b], tsems[b]).wait()

        def consume(b, accs):
            def body(g, accs):
                new = list(accs)
                i = 0
                for dr in range(_GR):
                    for cv in range(vpr):
                        idx = pl.ds(cv * _L, _L)
                        o = obuf[b, g, dr, idx]
                        t = tbuf[b, g, dr, idx]
                        d = o - t
                        sq = d * d
                        a = i % _NACC
                        new[a] = new[a] + jnp.where(t != 0.0, sq, 0.0)
                        i += 1
                return tuple(new)
            return plsc.parallel_loop(0, _GPC, step=1, carry=accs)(body)

        for b in range(_NBUF):
            start(b, b)

        zeros = jnp.zeros((_L,), jnp.float32)
        accs0 = (zeros,) * _NACC

        def outer(i, accs):
            for b in range(_NBUF):
                ci = i * _NBUF + b
                wait(b)
                accs = consume(b, accs)

                @pl.when(ci + _NBUF < nchunks)
                def _():
                    start(ci + _NBUF, b)
            return accs

        accs = lax.fori_loop(0, nchunks // _NBUF, outer, accs0)
        total = accs[0]
        for a in range(1, _NACC):
            total = total + accs[a]
        accv[...] = total
        pltpu.sync_copy(accv, out_hbm.at[wid])

    return masked_mse_partials


def kernel(output, target):
    nrows, ncols = output.shape
    partials = _build(nrows, ncols)(output, target)
    return jnp.sum(partials) / jnp.float32(output.size)


# speedup vs baseline: 1.0480x; 1.0480x over previous
import functools

import jax
import jax.numpy as jnp
from jax import lax
from jax.experimental import pallas as pl
from jax.experimental.pallas import tpu as pltpu
from jax.experimental.pallas import tpu_sc as plsc

_NC = 2
_NS = 16
_L = 16
_NW = _NC * _NS
_GR = 8
_GPC = 4
_CROWS = _GR * _GPC
_NBUF = 2
_NACC = 8

_SPLIT = 24576

_TC_BR = 512


@functools.lru_cache(maxsize=None)
def _build_sc(nrows: int, ncols: int, sc_rows: int):
    assert sc_rows % (_NW * _CROWS * _NBUF) == 0 and ncols % (8 * _L) == 0
    rpw = sc_rows // _NW
    nchunks = rpw // _CROWS
    vpr = ncols // _L
    mesh = plsc.VectorSubcoreMesh(core_axis_name="c", subcore_axis_name="s")

    @functools.partial(
        pl.kernel,
        out_type=jax.ShapeDtypeStruct((_NW, _L), jnp.float32),
        mesh=mesh,
        scratch_types=[
            pltpu.VMEM((_NBUF, _GPC, _GR, ncols), jnp.float32),
            pltpu.VMEM((_NBUF, _GPC, _GR, ncols), jnp.float32),
            pltpu.VMEM((_L,), jnp.float32),
            pltpu.SemaphoreType.DMA,
            pltpu.SemaphoreType.DMA,
            pltpu.SemaphoreType.DMA,
            pltpu.SemaphoreType.DMA,
        ],
    )
    def masked_mse_partials(o_hbm, t_hbm, out_hbm, obuf, tbuf, accv,
                            so0, so1, st0, st1):
        osems = (so0, so1)
        tsems = (st0, st1)
        wid = lax.axis_index("s") * _NC + lax.axis_index("c")
        base = wid * rpw

        o3 = o_hbm.reshape(nrows // _GR, _GR, ncols)
        t3 = t_hbm.reshape(nrows // _GR, _GR, ncols)

        def start(ci, b):
            grp = pl.multiple_of((base + ci * _CROWS) // _GR, _GPC)
            src = pl.ds(grp, _GPC)
            pltpu.async_copy(o3.at[src], obuf.at[b], osems[b])
            pltpu.async_copy(t3.at[src], tbuf.at[b], tsems[b])

        def wait(b):
            pltpu.make_async_copy(
                o3.at[pl.ds(0, _GPC)], obuf.at[b], osems[b]).wait()
            pltpu.make_async_copy(
                t3.at[pl.ds(0, _GPC)], tbuf.at[b], tsems[b]).wait()

        def consume(b, accs):
            def body(g, accs):
                new = list(accs)
                i = 0
                for dr in range(_GR):
                    for cv in range(vpr):
                        idx = pl.ds(cv * _L, _L)
                        o = obuf[b, g, dr, idx]
                        t = tbuf[b, g, dr, idx]
                        d = o - t
                        sq = d * d
                        a = i % _NACC
                        new[a] = new[a] + jnp.where(t != 0.0, sq, 0.0)
                        i += 1
                return tuple(new)
            return plsc.parallel_loop(0, _GPC, step=1, carry=accs)(body)

        for b in range(_NBUF):
            start(b, b)

        zeros = jnp.zeros((_L,), jnp.float32)
        accs0 = (zeros,) * _NACC

        def outer(i, accs):
            for b in range(_NBUF):
                ci = i * _NBUF + b
                wait(b)
                accs = consume(b, accs)

                @pl.when(ci + _NBUF < nchunks)
                def _():
                    start(ci + _NBUF, b)
            return accs

        accs = lax.fori_loop(0, nchunks // _NBUF, outer, accs0)
        total = accs[0]
        for a in range(1, _NACC):
            total = total + accs[a]
        accv[...] = total
        pltpu.sync_copy(accv, out_hbm.at[wid])

    return masked_mse_partials


def _tc_body(o_ref, t_ref, out_ref):
    i = pl.program_id(0)

    @pl.when(i == 0)
    def _():
        out_ref[...] = jnp.zeros_like(out_ref)

    o = o_ref[...]
    t = t_ref[...]
    d = o - t
    sq = jnp.where(t != 0.0, d * d, 0.0)
    br, ncols = o.shape
    part = jnp.sum(sq.reshape(br // 8, 8, ncols // 128, 128), axis=(0, 2))
    out_ref[...] += part


@functools.lru_cache(maxsize=None)
def _build_tc(nrows: int, ncols: int, row_lo: int):
    tc_rows = nrows - row_lo
    assert tc_rows % _TC_BR == 0 and row_lo % _TC_BR == 0
    grid = tc_rows // _TC_BR
    blk0 = row_lo // _TC_BR
    return pl.pallas_call(
        _tc_body,
        grid=(grid,),
        in_specs=[
            pl.BlockSpec((_TC_BR, ncols), lambda i: (i + blk0, 0)),
            pl.BlockSpec((_TC_BR, ncols), lambda i: (i + blk0, 0)),
        ],
        out_specs=pl.BlockSpec((8, 128), lambda i: (0, 0)),
        out_shape=jax.ShapeDtypeStruct((8, 128), jnp.float32),
        compiler_params=pltpu.CompilerParams(
            dimension_semantics=("arbitrary",)),
    )


def kernel(output, target):
    nrows, ncols = output.shape
    sc_partials = _build_sc(nrows, ncols, _SPLIT)(output, target)
    tc_partial = _build_tc(nrows, ncols, _SPLIT)(output, target)
    total = jnp.sum(sc_partials) + jnp.sum(tc_partial)
    return total / jnp.float32(output.size)


# speedup vs baseline: 1.0551x; 1.0068x over previous
import functools

import jax
import jax.numpy as jnp
from jax import lax
from jax.experimental import pallas as pl
from jax.experimental.pallas import tpu as pltpu
from jax.experimental.pallas import tpu_sc as plsc

_NC = 2
_NS = 16
_L = 16
_NW = _NC * _NS
_GR = 8
_GPC = 4
_CROWS = _GR * _GPC
_NBUF = 2
_NACC = 8

_SPLIT = 24576

_TC_BR = 512


@functools.lru_cache(maxsize=None)
def _build_sc(nrows: int, ncols: int, sc_rows: int):
    assert sc_rows % (_NW * _CROWS * _NBUF) == 0 and ncols % (8 * _L) == 0
    rpw = sc_rows // _NW
    nchunks = rpw // _CROWS
    vpr = ncols // _L
    mesh = plsc.VectorSubcoreMesh(core_axis_name="c", subcore_axis_name="s")

    @functools.partial(
        pl.kernel,
        out_type=jax.ShapeDtypeStruct((_NW, _L), jnp.float32),
        mesh=mesh,
        cost_estimate=pl.CostEstimate(
            flops=3 * sc_rows * ncols,
            bytes_accessed=8 * sc_rows * ncols,
            transcendentals=0,
        ),
        scratch_types=[
            pltpu.VMEM((_NBUF, _GPC, _GR, ncols), jnp.float32),
            pltpu.VMEM((_NBUF, _GPC, _GR, ncols), jnp.float32),
            pltpu.VMEM((_L,), jnp.float32),
            pltpu.SemaphoreType.DMA,
            pltpu.SemaphoreType.DMA,
            pltpu.SemaphoreType.DMA,
            pltpu.SemaphoreType.DMA,
        ],
    )
    def masked_mse_partials(o_hbm, t_hbm, out_hbm, obuf, tbuf, accv,
                            so0, so1, st0, st1):
        osems = (so0, so1)
        tsems = (st0, st1)
        wid = lax.axis_index("s") * _NC + lax.axis_index("c")
        base = wid * rpw

        o3 = o_hbm.reshape(nrows // _GR, _GR, ncols)
        t3 = t_hbm.reshape(nrows // _GR, _GR, ncols)

        def start(ci, b):
            grp = pl.multiple_of((base + ci * _CROWS) // _GR, _GPC)
            src = pl.ds(grp, _GPC)
            pltpu.async_copy(o3.at[src], obuf.at[b], osems[b])
            pltpu.async_copy(t3.at[src], tbuf.at[b], tsems[b])

        def wait(b):
            pltpu.make_async_copy(
                o3.at[pl.ds(0, _GPC)], obuf.at[b], osems[b]).wait()
            pltpu.make_async_copy(
                t3.at[pl.ds(0, _GPC)], tbuf.at[b], tsems[b]).wait()

        def consume(b, accs):
            def body(g, accs):
                new = list(accs)
                i = 0
                for dr in range(_GR):
                    for cv in range(vpr):
                        idx = pl.ds(cv * _L, _L)
                        o = obuf[b, g, dr, idx]
                        t = tbuf[b, g, dr, idx]
                        d = o - t
                        sq = d * d
                        a = i % _NACC
                        new[a] = new[a] + jnp.where(t != 0.0, sq, 0.0)
                        i += 1
                return tuple(new)
            return plsc.parallel_loop(0, _GPC, step=1, carry=accs)(body)

        for b in range(_NBUF):
            start(b, b)

        zeros = jnp.zeros((_L,), jnp.float32)
        accs0 = (zeros,) * _NACC

        def outer(i, accs):
            for b in range(_NBUF):
                ci = i * _NBUF + b
                wait(b)
                accs = consume(b, accs)

                @pl.when(ci + _NBUF < nchunks)
                def _():
                    start(ci + _NBUF, b)
            return accs

        accs = lax.fori_loop(0, nchunks // _NBUF, outer, accs0)
        total = accs[0]
        for a in range(1, _NACC):
            total = total + accs[a]
        accv[...] = total
        pltpu.sync_copy(accv, out_hbm.at[wid])

    return masked_mse_partials


def _tc_body(o_ref, t_ref, out_ref):
    i = pl.program_id(0)

    @pl.when(i == 0)
    def _():
        out_ref[...] = jnp.zeros_like(out_ref)

    o = o_ref[...]
    t = t_ref[...]
    d = o - t
    sq = jnp.where(t != 0.0, d * d, 0.0)
    br, ncols = o.shape
    part = jnp.sum(sq.reshape(br // 8, 8, ncols // 128, 128), axis=(0, 2))
    out_ref[...] += part


@functools.lru_cache(maxsize=None)
def _build_tc(nrows: int, ncols: int, row_lo: int):
    tc_rows = nrows - row_lo
    assert tc_rows % _TC_BR == 0 and row_lo % _TC_BR == 0
    grid = tc_rows // _TC_BR
    blk0 = row_lo // _TC_BR
    return pl.pallas_call(
        _tc_body,
        grid=(grid,),
        in_specs=[
            pl.BlockSpec((_TC_BR, ncols), lambda i: (i + blk0, 0)),
            pl.BlockSpec((_TC_BR, ncols), lambda i: (i + blk0, 0)),
        ],
        out_specs=pl.BlockSpec((8, 128), lambda i: (0, 0)),
        out_shape=jax.ShapeDtypeStruct((8, 128), jnp.float32),
        compiler_params=pltpu.CompilerParams(
            dimension_semantics=("arbitrary",)),
        cost_estimate=pl.CostEstimate(
            flops=3 * tc_rows * ncols,
            bytes_accessed=8 * tc_rows * ncols,
            transcendentals=0,
        ),
    )


def kernel(output, target):
    nrows, ncols = output.shape
    tc_partial = _build_tc(nrows, ncols, _SPLIT)(output, target)
    sc_partials = _build_sc(nrows, ncols, _SPLIT)(output, target)
    total = jnp.sum(sc_partials) + jnp.sum(tc_partial)
    return total / jnp.float32(output.size)
